# Initial kernel scaffold; baseline (speedup 1.0000x reference)
import jax
import jax.numpy as jnp
from jax.experimental import pallas as pl


def kernel(context, spans):
    raise NotImplementedError("write your pallas kernel here")


# speedup vs baseline: 33.8581x; 33.8581x over previous
import functools

import jax
import jax.numpy as jnp
from jax.experimental import pallas as pl
from jax.experimental.pallas import tpu as pltpu

_S = 2048
_D = 768
_NB = 256
_LEVELS = 8
_NEG_ROW = _LEVELS * _NB


def _body(ctx_ref, b_ref, e_ref, a1_ref, a2_ref, t1_ref, t2_ref,
          out_ref, tab_ref, *, nq):
    neg = jnp.float32(float("-inf"))

    tab_ref[pl.ds(_NEG_ROW, 8), :] = jnp.full((8, _D), neg, jnp.float32)

    def bm_step(j, _):
        rows = ctx_ref[0, pl.ds(j * 8, 8), :]
        tab_ref[pl.ds(j, 1), :] = jnp.max(rows, axis=0, keepdims=True)
        return 0
    jax.lax.fori_loop(0, _NB, bm_step, 0)

    for k in range(1, _LEVELS):
        h = 1 << (k - 1)
        prev = (k - 1) * _NB
        cur = k * _NB
        tab_ref[pl.ds(cur, _NB), :] = jnp.maximum(
            tab_ref[pl.ds(prev, _NB), :],
            tab_ref[pl.ds(prev + h, _NB), :])

    rid = jax.lax.broadcasted_iota(jnp.int32, (8, _D), 0)

    def q_step(g, _):
        rows_out = []
        for r in range(8):
            i = g * 8 + r
            b0 = b_ref[0, 0, i]
            e0 = e_ref[0, 0, i]
            a1 = pl.multiple_of(a1_ref[0, 0, i], 8)
            a2 = pl.multiple_of(a2_ref[0, 0, i], 8)
            x1 = t1_ref[0, 0, i]
            x2 = t2_ref[0, 0, i]
            cL = ctx_ref[0, pl.ds(a1, 8), :]
            cR = ctx_ref[0, pl.ds(a2, 8), :]
            mL = (rid + a1 >= b0) & (rid + a1 < e0)
            mR = (rid + a2 >= b0) & (rid + a2 < e0)
            eL = jnp.max(jnp.where(mL, cL, neg), axis=0, keepdims=True)
            eR = jnp.max(jnp.where(mR, cR, neg), axis=0, keepdims=True)
            tL = tab_ref[pl.ds(x1, 1), :]
            tR = tab_ref[pl.ds(x2, 1), :]
            rows_out.append(jnp.maximum(jnp.maximum(eL, eR),
                                        jnp.maximum(tL, tR)))
        out_ref[0, pl.ds(g * 8, 8), :] = jnp.concatenate(rows_out, axis=0)
        return 0
    jax.lax.fori_loop(0, nq // 8, q_step, 0)


@jax.jit
def kernel(context, spans):
    B, S, D = context.shape
    n = spans.shape[1]
    nq = (n + 7) // 8 * 8

    b = spans[..., 0].astype(jnp.int32)
    e = spans[..., 1].astype(jnp.int32)
    e = jnp.where(e == b, b + 1, e)
    pad = nq - n
    b = jnp.pad(b, ((0, 0), (0, pad)))
    e = jnp.pad(e, ((0, 0), (0, pad)), constant_values=1)

    c1 = b >> 3
    c2 = (e - 1) >> 3
    il = c2 - c1 - 1
    k = 31 - jax.lax.clz(jnp.maximum(il, 1))
    has = il > 0
    t1 = jnp.where(has, (k << 8) + c1 + 1, _NEG_ROW)
    t2 = jnp.where(has, (k << 8) + c2 - (1 << k), _NEG_ROW)

    def s3(x):
        return x.reshape(B, 1, nq)

    smem = pl.BlockSpec((1, 1, nq), lambda i: (i, 0, 0),
                        memory_space=pltpu.SMEM)
    out = pl.pallas_call(
        functools.partial(_body, nq=nq),
        grid=(B,),
        in_specs=[
            pl.BlockSpec((1, S, D), lambda i: (i, 0, 0)),
            smem, smem, smem, smem, smem, smem,
        ],
        out_specs=pl.BlockSpec((1, nq, D), lambda i: (i, 0, 0)),
        out_shape=jax.ShapeDtypeStruct((B, nq, D), jnp.float32),
        scratch_shapes=[pltpu.VMEM((_LEVELS * _NB + 8, D), jnp.float32)],
    )(context, s3(b), s3(e), s3(c1 << 3), s3(c2 << 3), s3(t1), s3(t2))
    return out[:, :n, :]
